# baseline (speedup 1.0000x reference)
import functools

import jax
import jax.numpy as jnp
from jax import lax
from jax.experimental import pallas as pl
from jax.experimental.pallas import tpu as pltpu
from jax.experimental.pallas import tpu_sc as plsc

NUM_FIELDS = 26
EMB_DIM = 32
NC = 2
NS = 16
NW = NC * NS


def _make_gather(B, H, F, V):
    R = B * H
    per_w = R // NW
    mesh = plsc.VectorSubcoreMesh(
        core_axis_name="c", subcore_axis_name="s", num_cores=NC, num_subcores=NS
    )

    @functools.partial(
        pl.kernel,
        out_type=jax.ShapeDtypeStruct((R, F * EMB_DIM), jnp.float32),
        mesh=mesh,
        scratch_types=[
            pltpu.VMEM((per_w,), jnp.int32),
            pltpu.VMEM((per_w,), jnp.int32),
            pltpu.VMEM((per_w, EMB_DIM), jnp.float32),
            pltpu.VMEM((per_w, EMB_DIM), jnp.float32),
            pltpu.SemaphoreType.DMA,
            pltpu.SemaphoreType.DMA,
            pltpu.SemaphoreType.DMA,
            pltpu.SemaphoreType.DMA,
        ],
        compiler_params=pltpu.CompilerParams(use_tc_tiling_on_sc=False),
    )
    def gather_kernel(*refs):
        tabs = refs[:F]
        idxT_hbm, out_hbm = refs[F], refs[F + 1]
        idx0, idx1, rows0, rows1, gsem0, gsem1, wsem0, wsem1 = refs[F + 2:]
        ibufs = (idx0, idx1)
        rbufs = (rows0, rows1)
        gsems = (gsem0, gsem1)
        wsems = (wsem0, wsem1)

        wid = lax.axis_index("s") * NC + lax.axis_index("c")
        base = wid * per_w

        def load_idx(f):
            pltpu.sync_copy(idxT_hbm.at[f].at[pl.ds(base, per_w)], ibufs[f % 2])

        def gather(f):
            p = f % 2
            pltpu.async_copy(tabs[f].at[ibufs[p]], rbufs[p], gsems[p])

        def gather_wait(f):
            p = f % 2
            pltpu.make_async_copy(
                tabs[f].at[ibufs[p]], rbufs[p], gsems[p]).wait()

        def writeback(f):
            p = f % 2
            pltpu.async_copy(
                rbufs[p],
                out_hbm.at[pl.ds(base, per_w), pl.ds(f * EMB_DIM, EMB_DIM)],
                wsems[p])

        def writeback_wait(f):
            p = f % 2
            pltpu.make_async_copy(
                rbufs[p],
                out_hbm.at[pl.ds(base, per_w), pl.ds(0, EMB_DIM)],
                wsems[p]).wait()

        load_idx(0)
        gather(0)
        for f in range(F):
            if f + 1 < F:
                if f - 1 >= 0:
                    writeback_wait(f - 1)
                load_idx(f + 1)
                gather(f + 1)
            gather_wait(f)
            writeback(f)
        writeback_wait(F - 2)
        writeback_wait(F - 1)

    return gather_kernel


def kernel(Xc, tables):
    B, H, F = Xc.shape
    V = tables.shape[1]
    idxT = Xc.reshape(B * H, F).T.astype(jnp.int32)
    tabs = [tables[f] for f in range(F)]
    out2d = _make_gather(B, H, F, V)(*tabs, idxT)
    return out2d.reshape(B, H, F * EMB_DIM)


# speedup vs baseline: 2.0053x; 2.0053x over previous
import functools

import jax
import jax.numpy as jnp
from jax import lax
from jax.experimental import pallas as pl
from jax.experimental.pallas import tpu as pltpu
from jax.experimental.pallas import tpu_sc as plsc

NUM_FIELDS = 26
EMB_DIM = 32
NC = 2
NS = 16
NW = NC * NS

HN = 5


def _make_lookup(B, H, F, V):
    assert H % HN == 0
    n_chunks = H // HN
    mesh = plsc.VectorSubcoreMesh(
        core_axis_name="c", subcore_axis_name="s", num_cores=NC, num_subcores=NS
    )

    @functools.partial(
        pl.kernel,
        out_type=jax.ShapeDtypeStruct((H, F * EMB_DIM, B), jnp.float32),
        mesh=mesh,
        scratch_types=[
            pltpu.VMEM((V,), jnp.float32),
            pltpu.VMEM((HN, B), jnp.int32),
            pltpu.VMEM((HN, B), jnp.int32),
            pltpu.VMEM((HN, 1, B), jnp.float32),
            pltpu.VMEM((HN, 1, B), jnp.float32),
            pltpu.SemaphoreType.DMA,
            pltpu.SemaphoreType.DMA,
            pltpu.SemaphoreType.DMA,
            pltpu.SemaphoreType.DMA,
        ],
        compiler_params=pltpu.CompilerParams(use_tc_tiling_on_sc=False, needs_layout_passes=False),
    )
    def lookup_kernel(t_hbm, idx_hbm, out_hbm,
                      rowbuf, idx0, idx1, ob0, ob1, is0, is1, os0, os1):
        e = lax.axis_index("s") * NC + lax.axis_index("c")
        ibufs = (idx0, idx1)
        obufs = (ob0, ob1)
        isems = (is0, is1)
        osems = (os0, os1)

        def compute_chunk(p):
            ibuf, obuf = ibufs[p], obufs[p]

            def hbody(h, carry):
                row_idx = ibuf.at[h]
                row_out_view = obuf.at[h]
                for b0 in range(0, B, 16):
                    v = row_idx[pl.ds(b0, 16)]
                    row_out_view[0, pl.ds(b0, 16)] = plsc.load_gather(
                        rowbuf, [v])
                return carry

            lax.fori_loop(0, HN, hbody, 0)

        def fbody(f, carry):
            fe = f * EMB_DIM + e
            pltpu.sync_copy(t_hbm.at[fe], rowbuf)

            pltpu.async_copy(idx_hbm.at[f].at[pl.ds(0, HN), :], ibufs[0],
                             isems[0])
            for c in range(n_chunks):
                p = c % 2
                if c + 1 < n_chunks:
                    pltpu.async_copy(
                        idx_hbm.at[f].at[pl.ds((c + 1) * HN, HN), :],
                        ibufs[1 - p], isems[1 - p])
                pltpu.make_async_copy(
                    idx_hbm.at[f].at[pl.ds(0, HN), :], ibufs[p],
                    isems[p]).wait()
                if c >= 2:
                    pltpu.make_async_copy(
                        obufs[p],
                        out_hbm.at[pl.ds(0, HN), pl.ds(fe, 1), :],
                        osems[p]).wait()
                compute_chunk(p)
                pltpu.async_copy(
                    obufs[p],
                    out_hbm.at[pl.ds(c * HN, HN), pl.ds(fe, 1), :],
                    osems[p])
            for p in range(2):
                pltpu.make_async_copy(
                    obufs[p], out_hbm.at[pl.ds(0, HN), pl.ds(fe, 1), :],
                    osems[p]).wait()
            return carry

        lax.fori_loop(0, F, fbody, 0)

    return lookup_kernel


def kernel(Xc, tables):
    B, H, F = Xc.shape
    V = tables.shape[1]
    t = tables.transpose(0, 2, 1).reshape(F * EMB_DIM, V)
    idxT = Xc.transpose(2, 1, 0).astype(jnp.int32)
    out_t = _make_lookup(B, H, F, V)(t, idxT)
    return out_t.transpose(2, 0, 1)


# speedup vs baseline: 2.0066x; 1.0007x over previous
import functools

import jax
import jax.numpy as jnp
from jax import lax
from jax.experimental import pallas as pl
from jax.experimental.pallas import tpu as pltpu
from jax.experimental.pallas import tpu_sc as plsc

NUM_FIELDS = 26
EMB_DIM = 32
NC = 2
NS = 16
NW = NC * NS

HN = 5


def _make_lookup(B, H, F, V):
    assert H % HN == 0
    n_chunks = H // HN
    mesh = plsc.VectorSubcoreMesh(
        core_axis_name="c", subcore_axis_name="s", num_cores=NC, num_subcores=NS
    )

    @functools.partial(
        pl.kernel,
        out_type=jax.ShapeDtypeStruct((H, F * EMB_DIM, B), jnp.float32),
        mesh=mesh,
        scratch_types=[
            pltpu.VMEM((V,), jnp.float32),
            pltpu.VMEM((HN, B), jnp.int32),
            pltpu.VMEM((HN, B), jnp.int32),
            pltpu.VMEM((HN, 1, B), jnp.float32),
            pltpu.VMEM((HN, 1, B), jnp.float32),
            pltpu.SemaphoreType.DMA,
            pltpu.SemaphoreType.DMA,
            pltpu.SemaphoreType.DMA,
            pltpu.SemaphoreType.DMA,
        ],
        compiler_params=pltpu.CompilerParams(use_tc_tiling_on_sc=False, needs_layout_passes=False),
    )
    def lookup_kernel(t_hbm, idx_hbm, out_hbm,
                      rowbuf, idx0, idx1, ob0, ob1, is0, is1, os0, os1):
        e = lax.axis_index("s") * NC + lax.axis_index("c")
        ibufs = (idx0, idx1)
        obufs = (ob0, ob1)
        isems = (is0, is1)
        osems = (os0, os1)

        def compute_chunk(p):
            ibuf, obuf = ibufs[p], obufs[p]

            def hbody(h, carry):
                row_idx = ibuf.at[h]
                row_out_view = obuf.at[h]
                for b0 in range(0, B, 16):
                    v = row_idx[pl.ds(b0, 16)]
                    row_out_view[0, pl.ds(b0, 16)] = plsc.load_gather(
                        rowbuf, [v])
                return carry

            lax.fori_loop(0, HN, hbody, 0)

        def fbody(f, carry):
            fe = f * EMB_DIM + e
            pltpu.sync_copy(t_hbm.at[f].at[e], rowbuf)

            pltpu.async_copy(idx_hbm.at[f].at[pl.ds(0, HN), :], ibufs[0],
                             isems[0])
            for c in range(n_chunks):
                p = c % 2
                if c + 1 < n_chunks:
                    pltpu.async_copy(
                        idx_hbm.at[f].at[pl.ds((c + 1) * HN, HN), :],
                        ibufs[1 - p], isems[1 - p])
                pltpu.make_async_copy(
                    idx_hbm.at[f].at[pl.ds(0, HN), :], ibufs[p],
                    isems[p]).wait()
                if c >= 2:
                    pltpu.make_async_copy(
                        obufs[p],
                        out_hbm.at[pl.ds(0, HN), pl.ds(fe, 1), :],
                        osems[p]).wait()
                compute_chunk(p)
                pltpu.async_copy(
                    obufs[p],
                    out_hbm.at[pl.ds(c * HN, HN), pl.ds(fe, 1), :],
                    osems[p])
            for p in range(2):
                pltpu.make_async_copy(
                    obufs[p], out_hbm.at[pl.ds(0, HN), pl.ds(fe, 1), :],
                    osems[p]).wait()
            return carry

        lax.fori_loop(0, F, fbody, 0)

    return lookup_kernel


def kernel(Xc, tables):
    B, H, F = Xc.shape
    V = tables.shape[1]
    t = tables.transpose(0, 2, 1)
    idxT = Xc.transpose(2, 1, 0).astype(jnp.int32)
    out_t = _make_lookup(B, H, F, V)(t, idxT)
    return out_t.transpose(2, 0, 1)


# speedup vs baseline: 2.0808x; 1.0370x over previous
import functools

import jax
import jax.numpy as jnp
from jax import lax
from jax.experimental import pallas as pl
from jax.experimental.pallas import tpu as pltpu
from jax.experimental.pallas import tpu_sc as plsc

NUM_FIELDS = 26
EMB_DIM = 32
NC = 2
NS = 16
NW = NC * NS

HN = 5


def _make_lookup(B, H, F, V):
    assert H % HN == 0
    n_chunks = H // HN
    mesh = plsc.VectorSubcoreMesh(
        core_axis_name="c", subcore_axis_name="s", num_cores=NC, num_subcores=NS
    )

    @functools.partial(
        pl.kernel,
        out_type=jax.ShapeDtypeStruct((H, F * EMB_DIM, B), jnp.float32),
        mesh=mesh,
        scratch_types=[
            pltpu.VMEM((V,), jnp.float32),
            pltpu.VMEM((HN, B), jnp.int32),
            pltpu.VMEM((HN, B), jnp.int32),
            pltpu.VMEM((HN, 1, B), jnp.float32),
            pltpu.VMEM((HN, 1, B), jnp.float32),
            pltpu.SemaphoreType.DMA,
            pltpu.SemaphoreType.DMA,
            pltpu.SemaphoreType.DMA,
            pltpu.SemaphoreType.DMA,
        ],
        compiler_params=pltpu.CompilerParams(use_tc_tiling_on_sc=False, needs_layout_passes=False),
    )
    def lookup_kernel(t_hbm, idx_hbm, out_hbm,
                      rowbuf, idx0, idx1, ob0, ob1, is0, is1, os0, os1):
        e = lax.axis_index("s") * NC + lax.axis_index("c")
        ibufs = (idx0, idx1)
        obufs = (ob0, ob1)
        isems = (is0, is1)
        osems = (os0, os1)

        def compute_chunk(p):
            ibuf, obuf = ibufs[p], obufs[p]

            def hbody(h, carry):
                row_idx = ibuf.at[h]
                row_out_view = obuf.at[h]
                for b0 in range(0, B, 16):
                    v = row_idx[pl.ds(b0, 16)]
                    row_out_view[0, pl.ds(b0, 16)] = plsc.load_gather(
                        rowbuf, [v])
                return carry

            lax.fori_loop(0, HN, hbody, 0)

        def fbody(f, carry):
            fe = f * EMB_DIM + e
            pltpu.sync_copy(t_hbm.at[f].at[e], rowbuf)

            pltpu.async_copy(idx_hbm.at[f].at[pl.ds(0, HN), :], ibufs[0],
                             isems[0])
            for c in range(n_chunks):
                p = c % 2
                if c + 1 < n_chunks:
                    pltpu.async_copy(
                        idx_hbm.at[f].at[pl.ds((c + 1) * HN, HN), :],
                        ibufs[1 - p], isems[1 - p])
                pltpu.make_async_copy(
                    idx_hbm.at[f].at[pl.ds(0, HN), :], ibufs[p],
                    isems[p]).wait()
                if c >= 2:
                    pltpu.make_async_copy(
                        obufs[p],
                        out_hbm.at[pl.ds(0, HN), pl.ds(fe, 1), :],
                        osems[p]).wait()
                compute_chunk(p)
                pltpu.async_copy(
                    obufs[p],
                    out_hbm.at[pl.ds(c * HN, HN), pl.ds(fe, 1), :],
                    osems[p])
            for p in range(2):
                pltpu.make_async_copy(
                    obufs[p], out_hbm.at[pl.ds(0, HN), pl.ds(fe, 1), :],
                    osems[p]).wait()
            return carry

        lax.fori_loop(0, F, fbody, 0)

    return lookup_kernel


def kernel(Xc, tables):
    B, H, F = Xc.shape
    V = tables.shape[1]
    t = jnp.pad(tables.transpose(0, 2, 1), ((0, 0), (0, 0), (0, (-V) % 128)))
    idxT = Xc.transpose(2, 1, 0).astype(jnp.int32)
    out_t = _make_lookup(B, H, F, t.shape[2])(t, idxT)
    return out_t.transpose(2, 0, 1)
